# Initial kernel scaffold; baseline (speedup 1.0000x reference)
import functools

import jax
import jax.numpy as jnp
from jax import lax
from jax.experimental import pallas as pl
from jax.experimental.pallas import tpu as pltpu
from jax.experimental.pallas import tpu_sc as plsc

N_NODES = 10000
N_EDGES = 320000
D = 128

NC = 2
NS = 16
NW = NC * NS
EDGES_PER_WORKER = N_EDGES // NW
CHUNK = 80
NCHUNKS = EDGES_PER_WORKER // CHUNK

ROW_BLK = 1000
GRID = N_NODES // ROW_BLK

_mesh = plsc.VectorSubcoreMesh(
    core_axis_name="c", subcore_axis_name="s", num_cores=NC, num_subcores=NS
)


def _sc_agg_body(with_deg, *refs):
    if with_deg:
        (y_hbm, src_hbm, dst_hbm, zrow_hbm, zdeg_hbm,
         agg_out, deg_out, src_v, dst_v, rows_v, ones_v, agg_sh, deg_sh,
         sem) = refs
    else:
        (y_hbm, src_hbm, dst_hbm, zrow_hbm,
         agg_out, src_v, dst_v, rows_v, agg_sh,
         sem) = refs

    c = lax.axis_index("c")
    s = lax.axis_index("s")
    wid = c * NS + s

    @pl.when(s < 10)
    def _zero():
        pltpu.sync_copy(zrow_hbm, agg_sh.at[pl.ds(s * 1000, 1000)])
        if with_deg:
            pltpu.sync_copy(zdeg_hbm, deg_sh.at[pl.ds(s * 1000, 1000)])

    if with_deg:
        for i in range(CHUNK // 16):
            ones_v[pl.ds(i * 16, 16)] = jnp.ones((16,), jnp.float32)

    plsc.subcore_barrier()

    base0 = wid * EDGES_PER_WORKER

    def body(i, carry):
        base = pl.multiple_of(base0 + i * CHUNK, 8)
        pltpu.sync_copy(src_hbm.at[pl.ds(base, CHUNK)], src_v)
        pltpu.sync_copy(dst_hbm.at[pl.ds(base, CHUNK)], dst_v)
        pltpu.async_copy(y_hbm.at[src_v], rows_v, sem).wait()
        pltpu.sync_copy(rows_v, agg_sh.at[dst_v], add=True)
        if with_deg:
            pltpu.sync_copy(ones_v, deg_sh.at[dst_v], add=True)
        return carry

    lax.fori_loop(0, NCHUNKS, body, 0)

    plsc.subcore_barrier()

    @pl.when(s < 10)
    def _writeback():
        pltpu.sync_copy(agg_sh.at[pl.ds(s * 1000, 1000)],
                        agg_out.at[c, pl.ds(s * 1000, 1000)])
        if with_deg:
            pltpu.sync_copy(deg_sh.at[pl.ds(s * 1000, 1000)],
                            deg_out.at[c, pl.ds(s * 1000, 1000)])


_sc_agg_deg = pl.kernel(
    functools.partial(_sc_agg_body, True),
    out_type=(
        jax.ShapeDtypeStruct((NC, N_NODES, D), jnp.float32),
        jax.ShapeDtypeStruct((NC, N_NODES), jnp.float32),
    ),
    mesh=_mesh,
    scratch_types=(
        pltpu.VMEM((CHUNK,), jnp.int32),
        pltpu.VMEM((CHUNK,), jnp.int32),
        pltpu.VMEM((CHUNK, D), jnp.float32),
        pltpu.VMEM((CHUNK,), jnp.float32),
        pltpu.VMEM_SHARED((N_NODES, D), jnp.float32),
        pltpu.VMEM_SHARED((N_NODES,), jnp.float32),
        pltpu.SemaphoreType.DMA,
    ),
)

_sc_agg = pl.kernel(
    functools.partial(_sc_agg_body, False),
    out_type=jax.ShapeDtypeStruct((NC, N_NODES, D), jnp.float32),
    mesh=_mesh,
    scratch_types=(
        pltpu.VMEM((CHUNK,), jnp.int32),
        pltpu.VMEM((CHUNK,), jnp.int32),
        pltpu.VMEM((CHUNK, D), jnp.float32),
        pltpu.VMEM_SHARED((N_NODES, D), jnp.float32),
        pltpu.SemaphoreType.DMA,
    ),
)


def _tc1_body(x_ref, wl_ref, wr_ref, b_ref, y_ref, z_ref):
    xb = x_ref[...]
    y_ref[...] = jnp.dot(xb, wl_ref[...], preferred_element_type=jnp.float32)
    z_ref[...] = (
        jnp.dot(xb, wr_ref[...], preferred_element_type=jnp.float32)
        + b_ref[...]
    )


def _tc2_body(agg_ref, deg_ref, z_ref, wl_ref, wr_ref, b_ref, y2_ref, z2_ref):
    agg = agg_ref[0] + agg_ref[1]
    deg = jnp.maximum(deg_ref[0] + deg_ref[1], 1.0)
    h = jnp.maximum(agg / deg + z_ref[...], 0.0)
    y2_ref[...] = jnp.dot(h, wl_ref[...], preferred_element_type=jnp.float32)
    z2_ref[...] = (
        jnp.dot(h, wr_ref[...], preferred_element_type=jnp.float32)
        + b_ref[...]
    )


def _tc3_body(agg_ref, deg_ref, z_ref, o_ref):
    agg = agg_ref[0] + agg_ref[1]
    deg = jnp.maximum(deg_ref[0] + deg_ref[1], 1.0)
    h = agg / deg + z_ref[...]
    m = jnp.max(h, axis=-1, keepdims=True)
    e = jnp.exp(h - m)
    lse = jnp.log(jnp.sum(e, axis=-1, keepdims=True)) + m
    o_ref[...] = h - lse


_row_spec = pl.BlockSpec((ROW_BLK, D), lambda i: (i, 0))
_w_spec = pl.BlockSpec((D, D), lambda i: (0, 0))
_b_spec = pl.BlockSpec((1, D), lambda i: (0, 0))
_agg_spec = pl.BlockSpec((NC, ROW_BLK, D), lambda i: (0, i, 0))
_deg_spec = pl.BlockSpec((NC, ROW_BLK, 1), lambda i: (0, i, 0))

_tc1 = pl.pallas_call(
    _tc1_body,
    grid=(GRID,),
    in_specs=[_row_spec, _w_spec, _w_spec, _b_spec],
    out_specs=[_row_spec, _row_spec],
    out_shape=[
        jax.ShapeDtypeStruct((N_NODES, D), jnp.float32),
        jax.ShapeDtypeStruct((N_NODES, D), jnp.float32),
    ],
)

_tc2 = pl.pallas_call(
    _tc2_body,
    grid=(GRID,),
    in_specs=[_agg_spec, _deg_spec, _row_spec, _w_spec, _w_spec, _b_spec],
    out_specs=[_row_spec, _row_spec],
    out_shape=[
        jax.ShapeDtypeStruct((N_NODES, D), jnp.float32),
        jax.ShapeDtypeStruct((N_NODES, D), jnp.float32),
    ],
)

_tc3 = pl.pallas_call(
    _tc3_body,
    grid=(GRID,),
    in_specs=[_agg_spec, _deg_spec, _row_spec],
    out_specs=_row_spec,
    out_shape=jax.ShapeDtypeStruct((N_NODES, D), jnp.float32),
)


def kernel(x, edge_index, W1_l, W1_r, b1, W2_l, W2_r, b2):
    src = edge_index[0].astype(jnp.int32)
    dst = edge_index[1].astype(jnp.int32)
    zrow = jnp.zeros((1000, D), jnp.float32)
    zdeg = jnp.zeros((1000,), jnp.float32)
    b1r = b1.reshape(1, D)
    b2r = b2.reshape(1, D)

    y1, z1 = _tc1(x, W1_l, W1_r, b1r)
    agg1, deg = _sc_agg_deg(y1, src, dst, zrow, zdeg)
    deg3 = deg.reshape(NC, N_NODES, 1)
    y2, z2 = _tc2(agg1, deg3, z1, W2_l, W2_r, b2r)
    agg2 = _sc_agg(y2, src, dst, zrow)
    out = _tc3(agg2, deg3, z2)
    return out


# speedup vs baseline: 5.4023x; 5.4023x over previous
import functools

import jax
import jax.numpy as jnp
from jax import lax
from jax.experimental import pallas as pl
from jax.experimental.pallas import tpu as pltpu
from jax.experimental.pallas import tpu_sc as plsc

N_NODES = 10000
N_EDGES = 320000
D = 128

NC = 2
NS = 16
NW = NC * NS
EDGES_PER_WORKER = N_EDGES // NW
CHUNK = 80
NCHUNKS = EDGES_PER_WORKER // CHUNK

ROW_BLK = 1000
GRID = N_NODES // ROW_BLK

_mesh = plsc.VectorSubcoreMesh(
    core_axis_name="c", subcore_axis_name="s", num_cores=NC, num_subcores=NS
)


def _sc_agg_body(with_deg, *refs):
    if with_deg:
        (y_hbm, src_hbm, dst_hbm, zrow_hbm, zdeg_hbm,
         agg_out, deg_out, src_v, dst_v, rows_v, ones_v, degstg_v, agg_sh,
         deg_sh, sem) = refs
    else:
        (y_hbm, src_hbm, dst_hbm, zrow_hbm,
         agg_out, src_v, dst_v, rows_v, agg_sh,
         sem) = refs

    c = lax.axis_index("c")
    s = lax.axis_index("s")
    wid = c * NS + s

    @pl.when(s < 10)
    def _zero():
        pltpu.sync_copy(zrow_hbm, agg_sh.at[pl.ds(s * 1000, 1000)])
        if with_deg:
            pltpu.sync_copy(zdeg_hbm, degstg_v)
            pltpu.sync_copy(degstg_v, deg_sh.at[pl.ds(s * 1000, 1000)])

    if with_deg:
        for i in range(CHUNK // 16):
            ones_v[pl.ds(i * 16, 16)] = jnp.ones((16,), jnp.float32)

    plsc.subcore_barrier()

    base0 = wid * EDGES_PER_WORKER

    def body(i, carry):
        base = pl.multiple_of(base0 + i * CHUNK, 8)
        pltpu.sync_copy(src_hbm.at[pl.ds(base, CHUNK)], src_v)
        pltpu.sync_copy(dst_hbm.at[pl.ds(base, CHUNK)], dst_v)
        pltpu.async_copy(y_hbm.at[src_v], rows_v, sem).wait()
        pltpu.sync_copy(rows_v, agg_sh.at[dst_v], add=True)
        if with_deg:
            pltpu.sync_copy(ones_v, deg_sh.at[dst_v], add=True)
        return carry

    lax.fori_loop(0, NCHUNKS, body, 0)

    plsc.subcore_barrier()

    @pl.when(s < 10)
    def _writeback():
        pltpu.sync_copy(agg_sh.at[pl.ds(s * 1000, 1000)],
                        agg_out.at[c, pl.ds(s * 1000, 1000)])
        if with_deg:
            off = pl.multiple_of(c * N_NODES + s * 1000, 8)
            pltpu.sync_copy(deg_sh.at[pl.ds(s * 1000, 1000)], degstg_v)
            pltpu.sync_copy(degstg_v, deg_out.at[pl.ds(off, 1000)])


_sc_agg_deg = pl.kernel(
    functools.partial(_sc_agg_body, True),
    out_type=(
        jax.ShapeDtypeStruct((NC, N_NODES, D), jnp.float32),
        jax.ShapeDtypeStruct((NC * N_NODES,), jnp.float32),
    ),
    mesh=_mesh,
    scratch_types=(
        pltpu.VMEM((CHUNK,), jnp.int32),
        pltpu.VMEM((CHUNK,), jnp.int32),
        pltpu.VMEM((CHUNK, D), jnp.float32),
        pltpu.VMEM((CHUNK,), jnp.float32),
        pltpu.VMEM((1000,), jnp.float32),
        pltpu.VMEM_SHARED((N_NODES, D), jnp.float32),
        pltpu.VMEM_SHARED((N_NODES,), jnp.float32),
        pltpu.SemaphoreType.DMA,
    ),
)

_sc_agg = pl.kernel(
    functools.partial(_sc_agg_body, False),
    out_type=jax.ShapeDtypeStruct((NC, N_NODES, D), jnp.float32),
    mesh=_mesh,
    scratch_types=(
        pltpu.VMEM((CHUNK,), jnp.int32),
        pltpu.VMEM((CHUNK,), jnp.int32),
        pltpu.VMEM((CHUNK, D), jnp.float32),
        pltpu.VMEM_SHARED((N_NODES, D), jnp.float32),
        pltpu.SemaphoreType.DMA,
    ),
)


def _tc1_body(x_ref, wl_ref, wr_ref, b_ref, y_ref, z_ref):
    xb = x_ref[...]
    y_ref[...] = jnp.dot(xb, wl_ref[...], preferred_element_type=jnp.float32)
    z_ref[...] = (
        jnp.dot(xb, wr_ref[...], preferred_element_type=jnp.float32)
        + b_ref[...]
    )


def _tc2_body(agg_ref, deg_ref, z_ref, wl_ref, wr_ref, b_ref, y2_ref, z2_ref):
    agg = agg_ref[0] + agg_ref[1]
    deg = jnp.maximum(deg_ref[0] + deg_ref[1], 1.0)
    h = jnp.maximum(agg / deg + z_ref[...], 0.0)
    y2_ref[...] = jnp.dot(h, wl_ref[...], preferred_element_type=jnp.float32)
    z2_ref[...] = (
        jnp.dot(h, wr_ref[...], preferred_element_type=jnp.float32)
        + b_ref[...]
    )


def _tc3_body(agg_ref, deg_ref, z_ref, o_ref):
    agg = agg_ref[0] + agg_ref[1]
    deg = jnp.maximum(deg_ref[0] + deg_ref[1], 1.0)
    h = agg / deg + z_ref[...]
    m = jnp.max(h, axis=-1, keepdims=True)
    e = jnp.exp(h - m)
    lse = jnp.log(jnp.sum(e, axis=-1, keepdims=True)) + m
    o_ref[...] = h - lse


_row_spec = pl.BlockSpec((ROW_BLK, D), lambda i: (i, 0))
_w_spec = pl.BlockSpec((D, D), lambda i: (0, 0))
_b_spec = pl.BlockSpec((1, D), lambda i: (0, 0))
_agg_spec = pl.BlockSpec((NC, ROW_BLK, D), lambda i: (0, i, 0))
_deg_spec = pl.BlockSpec((NC, ROW_BLK, 1), lambda i: (0, i, 0))

_tc1 = pl.pallas_call(
    _tc1_body,
    grid=(GRID,),
    in_specs=[_row_spec, _w_spec, _w_spec, _b_spec],
    out_specs=[_row_spec, _row_spec],
    out_shape=[
        jax.ShapeDtypeStruct((N_NODES, D), jnp.float32),
        jax.ShapeDtypeStruct((N_NODES, D), jnp.float32),
    ],
)

_tc2 = pl.pallas_call(
    _tc2_body,
    grid=(GRID,),
    in_specs=[_agg_spec, _deg_spec, _row_spec, _w_spec, _w_spec, _b_spec],
    out_specs=[_row_spec, _row_spec],
    out_shape=[
        jax.ShapeDtypeStruct((N_NODES, D), jnp.float32),
        jax.ShapeDtypeStruct((N_NODES, D), jnp.float32),
    ],
)

_tc3 = pl.pallas_call(
    _tc3_body,
    grid=(GRID,),
    in_specs=[_agg_spec, _deg_spec, _row_spec],
    out_specs=_row_spec,
    out_shape=jax.ShapeDtypeStruct((N_NODES, D), jnp.float32),
)


def kernel(x, edge_index, W1_l, W1_r, b1, W2_l, W2_r, b2):
    src = edge_index[0].astype(jnp.int32)
    dst = edge_index[1].astype(jnp.int32)
    zrow = jnp.zeros((1000, D), jnp.float32)
    zdeg = jnp.zeros((1000,), jnp.float32)
    b1r = b1.reshape(1, D)
    b2r = b2.reshape(1, D)

    y1, z1 = _tc1(x, W1_l, W1_r, b1r)
    agg1, deg = _sc_agg_deg(y1, src, dst, zrow, zdeg)
    deg3 = deg.reshape(NC, N_NODES, 1)
    y2, z2 = _tc2(agg1, deg3, z1, W2_l, W2_r, b2r)
    agg2 = _sc_agg(y2, src, dst, zrow)
    out = _tc3(agg2, deg3, z2)
    return out


# speedup vs baseline: 11.6632x; 2.1590x over previous
import functools

import jax
import jax.numpy as jnp
from jax import lax
from jax.experimental import pallas as pl
from jax.experimental.pallas import tpu as pltpu
from jax.experimental.pallas import tpu_sc as plsc

N_NODES = 10000
N_EDGES = 320000
D = 128

NC = 2
NS = 16
NW = NC * NS
EDGES_PER_WORKER = N_EDGES // NW
CHUNK = 80
NCHUNKS = EDGES_PER_WORKER // CHUNK

ROW_BLK = 1000
GRID = N_NODES // ROW_BLK

_mesh = plsc.VectorSubcoreMesh(
    core_axis_name="c", subcore_axis_name="s", num_cores=NC, num_subcores=NS
)


def _sc_agg_body(with_deg, *refs):
    if with_deg:
        (y_hbm, src_hbm, dst_hbm, zrow_hbm, zdeg_hbm,
         agg_out, deg_out, src_v, dst_v, rows0_v, rows1_v, ones_v, degstg_v,
         agg_sh, deg_sh, sem0, sem1) = refs
    else:
        (y_hbm, src_hbm, dst_hbm, zrow_hbm,
         agg_out, src_v, dst_v, rows0_v, rows1_v, agg_sh,
         sem0, sem1) = refs
    rows = (rows0_v, rows1_v)
    sems = (sem0, sem1)

    c = lax.axis_index("c")
    s = lax.axis_index("s")
    wid = c * NS + s

    @pl.when(s < 10)
    def _zero():
        pltpu.sync_copy(zrow_hbm, agg_sh.at[pl.ds(s * 1000, 1000)])
        if with_deg:
            pltpu.sync_copy(zdeg_hbm, degstg_v)
            pltpu.sync_copy(degstg_v, deg_sh.at[pl.ds(s * 1000, 1000)])

    if with_deg:
        for i in range(CHUNK // 16):
            ones_v[pl.ds(i * 16, 16)] = jnp.ones((16,), jnp.float32)

    pltpu.sync_copy(src_hbm.at[wid], src_v)
    pltpu.sync_copy(dst_hbm.at[wid], dst_v)

    plsc.subcore_barrier()

    def fire(chunk, buf):
        idx = src_v.at[pl.ds(pl.multiple_of(chunk * CHUNK, 8), CHUNK)]
        pltpu.async_copy(y_hbm.at[idx], rows[buf], sems[buf])

    def drain_and_scatter(chunk, buf):
        pltpu.make_async_copy(y_hbm.at[pl.ds(0, CHUNK)], rows[buf],
                              sems[buf]).wait()
        pltpu.sync_copy(rows[buf], agg_sh.at[dst_v.at[chunk]], add=True)
        if with_deg:
            pltpu.sync_copy(ones_v, deg_sh.at[dst_v.at[chunk]], add=True)

    assert NCHUNKS % 2 == 1
    fire(0, 0)

    def body(j, carry):
        fire(2 * j + 1, 1)
        drain_and_scatter(2 * j, 0)
        fire(2 * j + 2, 0)
        drain_and_scatter(2 * j + 1, 1)
        return carry

    lax.fori_loop(0, (NCHUNKS - 1) // 2, body, 0)
    drain_and_scatter(NCHUNKS - 1, 0)

    plsc.subcore_barrier()

    @pl.when(s < 10)
    def _writeback():
        pltpu.sync_copy(agg_sh.at[pl.ds(s * 1000, 1000)],
                        agg_out.at[c, pl.ds(s * 1000, 1000)])
        if with_deg:
            off = pl.multiple_of(c * N_NODES + s * 1000, 8)
            pltpu.sync_copy(deg_sh.at[pl.ds(s * 1000, 1000)], degstg_v)
            pltpu.sync_copy(degstg_v, deg_out.at[pl.ds(off, 1000)])


_sc_agg_deg = pl.kernel(
    functools.partial(_sc_agg_body, True),
    out_type=(
        jax.ShapeDtypeStruct((NC, N_NODES, D), jnp.float32),
        jax.ShapeDtypeStruct((NC * N_NODES,), jnp.float32),
    ),
    mesh=_mesh,
    scratch_types=(
        pltpu.VMEM((EDGES_PER_WORKER,), jnp.int32),
        pltpu.VMEM((NCHUNKS, CHUNK), jnp.int32),
        pltpu.VMEM((CHUNK, D), jnp.float32),
        pltpu.VMEM((CHUNK, D), jnp.float32),
        pltpu.VMEM((CHUNK,), jnp.float32),
        pltpu.VMEM((1000,), jnp.float32),
        pltpu.VMEM_SHARED((N_NODES, D), jnp.float32),
        pltpu.VMEM_SHARED((N_NODES,), jnp.float32),
        pltpu.SemaphoreType.DMA,
        pltpu.SemaphoreType.DMA,
    ),
)

_sc_agg = pl.kernel(
    functools.partial(_sc_agg_body, False),
    out_type=jax.ShapeDtypeStruct((NC, N_NODES, D), jnp.float32),
    mesh=_mesh,
    scratch_types=(
        pltpu.VMEM((EDGES_PER_WORKER,), jnp.int32),
        pltpu.VMEM((NCHUNKS, CHUNK), jnp.int32),
        pltpu.VMEM((CHUNK, D), jnp.float32),
        pltpu.VMEM((CHUNK, D), jnp.float32),
        pltpu.VMEM_SHARED((N_NODES, D), jnp.float32),
        pltpu.SemaphoreType.DMA,
        pltpu.SemaphoreType.DMA,
    ),
)


def _tc1_body(x_ref, wl_ref, wr_ref, b_ref, y_ref, z_ref):
    xb = x_ref[...]
    y_ref[...] = jnp.dot(xb, wl_ref[...], preferred_element_type=jnp.float32)
    z_ref[...] = (
        jnp.dot(xb, wr_ref[...], preferred_element_type=jnp.float32)
        + b_ref[...]
    )


def _tc2_body(agg_ref, deg_ref, z_ref, wl_ref, wr_ref, b_ref, y2_ref, z2_ref):
    agg = agg_ref[0] + agg_ref[1]
    deg = jnp.maximum(deg_ref[0] + deg_ref[1], 1.0)
    h = jnp.maximum(agg / deg + z_ref[...], 0.0)
    y2_ref[...] = jnp.dot(h, wl_ref[...], preferred_element_type=jnp.float32)
    z2_ref[...] = (
        jnp.dot(h, wr_ref[...], preferred_element_type=jnp.float32)
        + b_ref[...]
    )


def _tc3_body(agg_ref, deg_ref, z_ref, o_ref):
    agg = agg_ref[0] + agg_ref[1]
    deg = jnp.maximum(deg_ref[0] + deg_ref[1], 1.0)
    h = agg / deg + z_ref[...]
    m = jnp.max(h, axis=-1, keepdims=True)
    e = jnp.exp(h - m)
    lse = jnp.log(jnp.sum(e, axis=-1, keepdims=True)) + m
    o_ref[...] = h - lse


_row_spec = pl.BlockSpec((ROW_BLK, D), lambda i: (i, 0))
_w_spec = pl.BlockSpec((D, D), lambda i: (0, 0))
_b_spec = pl.BlockSpec((1, D), lambda i: (0, 0))
_agg_spec = pl.BlockSpec((NC, ROW_BLK, D), lambda i: (0, i, 0))
_deg_spec = pl.BlockSpec((NC, ROW_BLK, 1), lambda i: (0, i, 0))

_tc1 = pl.pallas_call(
    _tc1_body,
    grid=(GRID,),
    in_specs=[_row_spec, _w_spec, _w_spec, _b_spec],
    out_specs=[_row_spec, _row_spec],
    out_shape=[
        jax.ShapeDtypeStruct((N_NODES, D), jnp.float32),
        jax.ShapeDtypeStruct((N_NODES, D), jnp.float32),
    ],
)

_tc2 = pl.pallas_call(
    _tc2_body,
    grid=(GRID,),
    in_specs=[_agg_spec, _deg_spec, _row_spec, _w_spec, _w_spec, _b_spec],
    out_specs=[_row_spec, _row_spec],
    out_shape=[
        jax.ShapeDtypeStruct((N_NODES, D), jnp.float32),
        jax.ShapeDtypeStruct((N_NODES, D), jnp.float32),
    ],
)

_tc3 = pl.pallas_call(
    _tc3_body,
    grid=(GRID,),
    in_specs=[_agg_spec, _deg_spec, _row_spec],
    out_specs=_row_spec,
    out_shape=jax.ShapeDtypeStruct((N_NODES, D), jnp.float32),
)


def kernel(x, edge_index, W1_l, W1_r, b1, W2_l, W2_r, b2):
    src = edge_index[0].astype(jnp.int32).reshape(NW, EDGES_PER_WORKER)
    dst = edge_index[1].astype(jnp.int32).reshape(NW, NCHUNKS, CHUNK)
    zrow = jnp.zeros((1000, D), jnp.float32)
    zdeg = jnp.zeros((1000,), jnp.float32)
    b1r = b1.reshape(1, D)
    b2r = b2.reshape(1, D)

    y1, z1 = _tc1(x, W1_l, W1_r, b1r)
    agg1, deg = _sc_agg_deg(y1, src, dst, zrow, zdeg)
    deg3 = deg.reshape(NC, N_NODES, 1)
    y2, z2 = _tc2(agg1, deg3, z1, W2_l, W2_r, b2r)
    agg2 = _sc_agg(y2, src, dst, zrow)
    out = _tc3(agg2, deg3, z2)
    return out


# speedup vs baseline: 13.3605x; 1.1455x over previous
import functools

import jax
import jax.numpy as jnp
from jax import lax
from jax.experimental import pallas as pl
from jax.experimental.pallas import tpu as pltpu
from jax.experimental.pallas import tpu_sc as plsc

N_NODES = 10000
N_EDGES = 320000
D = 128

NC = 2
NS = 16
NW = NC * NS
EDGES_PER_WORKER = N_EDGES // NW
CHUNK = 80
NCHUNKS = EDGES_PER_WORKER // CHUNK

ROW_BLK = 1000
GRID = N_NODES // ROW_BLK

_mesh = plsc.VectorSubcoreMesh(
    core_axis_name="c", subcore_axis_name="s", num_cores=NC, num_subcores=NS
)


NB = 3


def _sc_agg_body(with_deg, *refs):
    if with_deg:
        (y_hbm, pk_hbm, zrow_hbm, zdeg_hbm,
         agg_out, deg_out,
         pk_v, sb0, sb1, sb2, db0, db1, db2, r0, r1, r2, ones_v, degstg_v,
         agg_sh, deg_sh, g0, g1, g2, s0, s1, s2) = refs
    else:
        (y_hbm, pk_hbm, zrow_hbm,
         agg_out,
         pk_v, sb0, sb1, sb2, db0, db1, db2, r0, r1, r2,
         agg_sh, g0, g1, g2, s0, s1, s2) = refs
    srcb = (sb0, sb1, sb2)
    dstb = (db0, db1, db2)
    rows = (r0, r1, r2)
    gsem = (g0, g1, g2)
    ssem = (s0, s1, s2)

    c = lax.axis_index("c")
    s = lax.axis_index("s")
    wid = c * NS + s

    @pl.when(s < 10)
    def _zero():
        pltpu.sync_copy(zrow_hbm, agg_sh.at[pl.ds(s * 1000, 1000)])
        if with_deg:
            pltpu.sync_copy(zdeg_hbm, degstg_v)
            pltpu.sync_copy(degstg_v, deg_sh.at[pl.ds(s * 1000, 1000)])

    if with_deg:
        for i in range(CHUNK // 16):
            ones_v[pl.ds(i * 16, 16)] = jnp.ones((16,), jnp.float32)

    pltpu.sync_copy(pk_hbm.at[wid], pk_v)

    plsc.subcore_barrier()

    def unpack(chunk, b):
        base = chunk * CHUNK
        for k in range(CHUNK // 16):
            p = pk_v[pl.ds(base + 16 * k, 16)]
            srcb[b][pl.ds(16 * k, 16)] = p & 0xFFFF
            dstb[b][pl.ds(16 * k, 16)] = lax.shift_right_logical(p, 16)

    def fire_gather(chunk, b):
        unpack(chunk, b)
        pltpu.async_copy(y_hbm.at[srcb[b]], rows[b], gsem[b])

    def wait_gather(b):
        pltpu.make_async_copy(y_hbm.at[pl.ds(0, CHUNK)], rows[b],
                              gsem[b]).wait()

    def fire_scatter(b):
        pltpu.async_copy(rows[b], agg_sh.at[dstb[b]], ssem[b], add=True)
        if with_deg:
            pltpu.async_copy(ones_v, deg_sh.at[dstb[b]], ssem[b], add=True)

    def wait_scatter(b):
        pltpu.make_async_copy(y_hbm.at[pl.ds(0, CHUNK)], rows[b],
                              ssem[b]).wait()
        if with_deg:
            pltpu.make_async_copy(zrow_hbm.at[0, pl.ds(0, CHUNK)], ones_v,
                                  ssem[b]).wait()

    assert NCHUNKS % NB == 2
    fire_gather(0, 0)

    def body(j, carry):
        for b in range(NB):
            i = NB * j + b
            b1 = (b + 1) % NB

            @pl.when(i >= 2)
            def _w():
                wait_scatter(b1)

            fire_gather(i + 1, b1)
            wait_gather(b)
            fire_scatter(b)
        return carry

    lax.fori_loop(0, NCHUNKS // NB, body, 0)
    wait_scatter(1)
    fire_gather(NCHUNKS - 1, 1)
    wait_gather(0)
    fire_scatter(0)
    wait_gather(1)
    fire_scatter(1)
    wait_scatter(2)
    wait_scatter(0)
    wait_scatter(1)

    plsc.subcore_barrier()

    @pl.when(s < 10)
    def _writeback():
        pltpu.sync_copy(agg_sh.at[pl.ds(s * 1000, 1000)],
                        agg_out.at[c, pl.ds(s * 1000, 1000)])
        if with_deg:
            off = pl.multiple_of(c * N_NODES + s * 1000, 8)
            pltpu.sync_copy(deg_sh.at[pl.ds(s * 1000, 1000)], degstg_v)
            pltpu.sync_copy(degstg_v, deg_out.at[pl.ds(off, 1000)])


_sc_agg_deg = pl.kernel(
    functools.partial(_sc_agg_body, True),
    out_type=(
        jax.ShapeDtypeStruct((NC, N_NODES, D), jnp.float32),
        jax.ShapeDtypeStruct((NC * N_NODES,), jnp.float32),
    ),
    mesh=_mesh,
    scratch_types=(
        pltpu.VMEM((EDGES_PER_WORKER,), jnp.int32),
        pltpu.VMEM((CHUNK,), jnp.int32),
        pltpu.VMEM((CHUNK,), jnp.int32),
        pltpu.VMEM((CHUNK,), jnp.int32),
        pltpu.VMEM((CHUNK,), jnp.int32),
        pltpu.VMEM((CHUNK,), jnp.int32),
        pltpu.VMEM((CHUNK,), jnp.int32),
        pltpu.VMEM((CHUNK, D), jnp.float32),
        pltpu.VMEM((CHUNK, D), jnp.float32),
        pltpu.VMEM((CHUNK, D), jnp.float32),
        pltpu.VMEM((CHUNK,), jnp.float32),
        pltpu.VMEM((1000,), jnp.float32),
        pltpu.VMEM_SHARED((N_NODES, D), jnp.float32),
        pltpu.VMEM_SHARED((N_NODES,), jnp.float32),
        pltpu.SemaphoreType.DMA,
        pltpu.SemaphoreType.DMA,
        pltpu.SemaphoreType.DMA,
        pltpu.SemaphoreType.DMA,
        pltpu.SemaphoreType.DMA,
        pltpu.SemaphoreType.DMA,
    ),
)

_sc_agg = pl.kernel(
    functools.partial(_sc_agg_body, False),
    out_type=jax.ShapeDtypeStruct((NC, N_NODES, D), jnp.float32),
    mesh=_mesh,
    scratch_types=(
        pltpu.VMEM((EDGES_PER_WORKER,), jnp.int32),
        pltpu.VMEM((CHUNK,), jnp.int32),
        pltpu.VMEM((CHUNK,), jnp.int32),
        pltpu.VMEM((CHUNK,), jnp.int32),
        pltpu.VMEM((CHUNK,), jnp.int32),
        pltpu.VMEM((CHUNK,), jnp.int32),
        pltpu.VMEM((CHUNK,), jnp.int32),
        pltpu.VMEM((CHUNK, D), jnp.float32),
        pltpu.VMEM((CHUNK, D), jnp.float32),
        pltpu.VMEM((CHUNK, D), jnp.float32),
        pltpu.VMEM_SHARED((N_NODES, D), jnp.float32),
        pltpu.SemaphoreType.DMA,
        pltpu.SemaphoreType.DMA,
        pltpu.SemaphoreType.DMA,
        pltpu.SemaphoreType.DMA,
        pltpu.SemaphoreType.DMA,
        pltpu.SemaphoreType.DMA,
    ),
)


def _tc1_body(x_ref, wl_ref, wr_ref, b_ref, y_ref, z_ref):
    xb = x_ref[...]
    y_ref[...] = jnp.dot(xb, wl_ref[...], preferred_element_type=jnp.float32)
    z_ref[...] = (
        jnp.dot(xb, wr_ref[...], preferred_element_type=jnp.float32)
        + b_ref[...]
    )


def _tc2_body(agg_ref, deg_ref, z_ref, wl_ref, wr_ref, b_ref, y2_ref, z2_ref):
    agg = agg_ref[0] + agg_ref[1]
    deg = jnp.maximum(deg_ref[0] + deg_ref[1], 1.0)
    h = jnp.maximum(agg / deg + z_ref[...], 0.0)
    y2_ref[...] = jnp.dot(h, wl_ref[...], preferred_element_type=jnp.float32)
    z2_ref[...] = (
        jnp.dot(h, wr_ref[...], preferred_element_type=jnp.float32)
        + b_ref[...]
    )


def _tc3_body(agg_ref, deg_ref, z_ref, o_ref):
    agg = agg_ref[0] + agg_ref[1]
    deg = jnp.maximum(deg_ref[0] + deg_ref[1], 1.0)
    h = agg / deg + z_ref[...]
    m = jnp.max(h, axis=-1, keepdims=True)
    e = jnp.exp(h - m)
    lse = jnp.log(jnp.sum(e, axis=-1, keepdims=True)) + m
    o_ref[...] = h - lse


_row_spec = pl.BlockSpec((ROW_BLK, D), lambda i: (i, 0))
_w_spec = pl.BlockSpec((D, D), lambda i: (0, 0))
_b_spec = pl.BlockSpec((1, D), lambda i: (0, 0))
_agg_spec = pl.BlockSpec((NC, ROW_BLK, D), lambda i: (0, i, 0))
_deg_spec = pl.BlockSpec((NC, ROW_BLK, 1), lambda i: (0, i, 0))

_tc1 = pl.pallas_call(
    _tc1_body,
    grid=(GRID,),
    in_specs=[_row_spec, _w_spec, _w_spec, _b_spec],
    out_specs=[_row_spec, _row_spec],
    out_shape=[
        jax.ShapeDtypeStruct((N_NODES, D), jnp.float32),
        jax.ShapeDtypeStruct((N_NODES, D), jnp.float32),
    ],
)

_tc2 = pl.pallas_call(
    _tc2_body,
    grid=(GRID,),
    in_specs=[_agg_spec, _deg_spec, _row_spec, _w_spec, _w_spec, _b_spec],
    out_specs=[_row_spec, _row_spec],
    out_shape=[
        jax.ShapeDtypeStruct((N_NODES, D), jnp.float32),
        jax.ShapeDtypeStruct((N_NODES, D), jnp.float32),
    ],
)

_tc3 = pl.pallas_call(
    _tc3_body,
    grid=(GRID,),
    in_specs=[_agg_spec, _deg_spec, _row_spec],
    out_specs=_row_spec,
    out_shape=jax.ShapeDtypeStruct((N_NODES, D), jnp.float32),
)


def kernel(x, edge_index, W1_l, W1_r, b1, W2_l, W2_r, b2):
    src = edge_index[0].astype(jnp.int32)
    dst = edge_index[1].astype(jnp.int32)
    pk = (src | (dst << 16)).reshape(NW, EDGES_PER_WORKER)
    zrow = jnp.zeros((1000, D), jnp.float32)
    zdeg = jnp.zeros((1000,), jnp.float32)
    b1r = b1.reshape(1, D)
    b2r = b2.reshape(1, D)

    y1, z1 = _tc1(x, W1_l, W1_r, b1r)
    agg1, deg = _sc_agg_deg(y1, pk, zrow, zdeg)
    deg3 = deg.reshape(NC, N_NODES, 1)
    y2, z2 = _tc2(agg1, deg3, z1, W2_l, W2_r, b2r)
    agg2 = _sc_agg(y2, pk, zrow)
    out = _tc3(agg2, deg3, z2)
    return out
